# Initial kernel scaffold; baseline (speedup 1.0000x reference)
import functools

import jax
import jax.numpy as jnp
from jax import lax
from jax.experimental import pallas as pl
from jax.experimental.pallas import tpu as pltpu
from jax.experimental.pallas import tpu_sc as plsc
import numpy as np

BBOX_XFORM_CLIP = float(np.log(1000.0 / 16.0))

N_BATCH = 8
N_ANC = 20000
K = 2000
L = 16
NV_ALL = N_ANC // L
NV_K = K // L
CAP = N_ANC + L
WCAP = 2048 + L
SIGN = jnp.int32(-2147483648)


def _zero_hist(hist):
    z = jnp.zeros((L,), jnp.int32)
    for g in range(16):
        hist[pl.ds(g * L, L)] = z


def _scalar_at(vec, lane, iota):
    return jnp.max(jnp.where(iota == lane, vec, 0))


def _popcount(mask):
    return jnp.max(plsc.all_reduce_population_count(mask))


def _find_digit(hist, k_rem, iota):
    g_sums = []
    for g in range(16):
        g_sums.append(jnp.sum(hist[pl.ds(g * L, L)]))
    suffix = [jnp.int32(0)] * 17
    for g in range(15, -1, -1):
        suffix[g] = suffix[g + 1] + g_sums[g]
    gstar = jnp.int32(0)
    ag = jnp.int32(0)
    for g in range(16):
        hit = jnp.logical_and(suffix[g] >= k_rem, suffix[g + 1] < k_rem)
        gstar = jnp.where(hit, g, gstar)
        ag = jnp.where(hit, suffix[g + 1], ag)
    hvec = hist[pl.ds(gstar * L, L)]
    rcum = plsc.cumsum(jnp.flip(hvec, 0))
    need = k_rem - ag
    j = jnp.max(plsc.all_reduce_ffs(rcum >= need))
    dloc = 15 - j
    cnt_d = _scalar_at(hvec, dloc, iota)
    rc_j = _scalar_at(rcum, j, iota)
    count_above = ag + (rc_j - cnt_d)
    return gstar * L + dloc, count_above


def _body(obj_hbm, anc_hbm, breg_hbm, out_hbm,
          stage, keys, idxs, win_k, win_i, tmp_k, tmp_i,
          hist, offs, gidx, rows_anc, rows_breg, out_tile, sem1, sem2):
    cid = lax.axis_index("c")
    sid = lax.axis_index("s")
    wid = sid * 2 + cid

    @pl.when(wid < N_BATCH)
    def _():
        b = wid
        iota = lax.iota(jnp.int32, L)
        zf = jnp.zeros((L,), jnp.int32)

        pltpu.sync_copy(obj_hbm.at[pl.ds(b * N_ANC, N_ANC)], stage)

        _zero_hist(hist)

        def h1(v, carry):
            f = stage[pl.ds(v * L, L)]
            bits = plsc.bitcast(f, jnp.int32)
            neg = bits < 0
            key = jnp.where(neg, ~bits, bits ^ SIGN)
            keys[pl.ds(v * L, L)] = key
            dig = lax.shift_right_logical(key, 24)
            cnt, last = plsc.scan_count(dig)
            plsc.addupdate_scatter(hist, [dig], cnt, mask=last)
            return carry

        lax.fori_loop(0, NV_ALL, h1, 0)

        k_rem = jnp.int32(K)
        bw = jnp.int32(0)
        n_cur = jnp.int32(N_ANC)
        for p in range(4):
            shift = 24 - 8 * p
            if p > 0:
                _zero_hist(hist)

                def hp(v, carry, shift=shift, n_cur=n_cur):
                    kv = keys[pl.ds(v * L, L)]
                    valid = (v * L + iota) < n_cur
                    dig = jnp.bitwise_and(
                        lax.shift_right_logical(kv, shift), 255)
                    cnt, last = plsc.scan_count(dig, mask=valid)
                    plsc.addupdate_scatter(hist, [dig], cnt, mask=last)
                    return carry

                lax.fori_loop(0, (n_cur + L - 1) // L, hp, 0)

            dstar, count_above = _find_digit(hist, k_rem, iota)

            def cp(v, carry, shift=shift, n_cur=n_cur, p=p, dstar=dstar):
                bw_, bs_ = carry
                kv = keys[pl.ds(v * L, L)]
                if p == 0:
                    iv = v * L + iota
                    valid_w = jnp.full((L,), True)
                else:
                    iv = idxs[pl.ds(v * L, L)]
                    valid_w = (v * L + iota) < n_cur
                dig = jnp.bitwise_and(
                    lax.shift_right_logical(kv, shift), 255).astype(jnp.int32)
                win = jnp.logical_and(dig > dstar, valid_w)
                sur = jnp.logical_and(dig == dstar, valid_w)
                plsc.store_compressed(win_k.at[pl.ds(bw_, L)], kv, mask=win)
                plsc.store_compressed(win_i.at[pl.ds(bw_, L)], iv, mask=win)
                plsc.store_compressed(keys.at[pl.ds(bs_, L)], kv, mask=sur)
                plsc.store_compressed(idxs.at[pl.ds(bs_, L)], iv, mask=sur)
                return bw_ + _popcount(win), bs_ + _popcount(sur)

            bw, n_cur = lax.fori_loop(
                0, (n_cur + L - 1) // L, cp, (bw, jnp.int32(0)))
            k_rem = k_rem - count_above

        def ap(v, carry):
            win_k[pl.ds(bw + v * L, L)] = keys[pl.ds(v * L, L)]
            win_i[pl.ds(bw + v * L, L)] = idxs[pl.ds(v * L, L)]
            return carry

        lax.fori_loop(0, (k_rem + L - 1) // L, ap, 0)

        def sort_pass(src_k, src_i, dst_k, dst_i, shift, comp_in):
            _zero_hist(hist)

            def sh(v, carry):
                kv = src_k[pl.ds(v * L, L)]
                if comp_in:
                    kv = ~kv
                dig = jnp.bitwise_and(lax.shift_right_logical(kv, shift), 255)
                cnt, last = plsc.scan_count(dig)
                plsc.addupdate_scatter(hist, [dig], cnt, mask=last)
                return carry

            lax.fori_loop(0, NV_K, sh, 0)

            carry = jnp.int32(0)
            for g in range(16):
                hv = hist[pl.ds(g * L, L)]
                c = plsc.cumsum(hv)
                offs[pl.ds(g * L, L)] = c - hv + carry
                carry = carry + jnp.max(c)

            def sp(v, carry2):
                kv = src_k[pl.ds(v * L, L)]
                if comp_in:
                    kv = ~kv
                iv = src_i[pl.ds(v * L, L)]
                dig = jnp.bitwise_and(lax.shift_right_logical(kv, shift), 255)
                cnt, last = plsc.scan_count(dig)
                base = plsc.load_gather(offs, [dig])
                addr = base + cnt - 1
                plsc.store_scatter(dst_k, [addr], kv)
                plsc.store_scatter(dst_i, [addr], iv)
                plsc.addupdate_scatter(offs, [dig], cnt, mask=last)
                return carry2

            lax.fori_loop(0, NV_K, sp, 0)

        sort_pass(win_k, win_i, tmp_k, tmp_i, 0, True)
        sort_pass(tmp_k, tmp_i, win_k, win_i, 8, False)
        sort_pass(win_k, win_i, tmp_k, tmp_i, 16, False)
        sort_pass(tmp_k, tmp_i, win_k, win_i, 24, False)

        def gi(v, carry):
            gidx[pl.ds(v * L, L)] = win_i[pl.ds(v * L, L)] + b * N_ANC
            return carry

        lax.fori_loop(0, NV_K, gi, 0)

        cp1 = pltpu.async_copy(anc_hbm.at[gidx], rows_anc, sem1)
        cp2 = pltpu.async_copy(breg_hbm.at[gidx], rows_breg, sem2)
        cp1.wait()
        cp2.wait()

        c0 = zf
        c1 = zf + 1
        c2 = zf + 2
        c3 = zf + 3
        c4 = zf + 4
        clip = jnp.full((L,), BBOX_XFORM_CLIP, jnp.float32)

        def dec(v, carry):
            r = v * L + iota
            ax1 = plsc.load_gather(rows_anc, [r, c0])
            ay1 = plsc.load_gather(rows_anc, [r, c1])
            ax2 = plsc.load_gather(rows_anc, [r, c2])
            ay2 = plsc.load_gather(rows_anc, [r, c3])
            dx = plsc.load_gather(rows_breg, [r, c0])
            dy = plsc.load_gather(rows_breg, [r, c1])
            dw = plsc.load_gather(rows_breg, [r, c2])
            dh = plsc.load_gather(rows_breg, [r, c3])
            w = ax2 - ax1 + 1.0
            h = ay2 - ay1 + 1.0
            ctr_x = ax1 + 0.5 * w
            ctr_y = ay1 + 0.5 * h
            dw_c = jnp.minimum(dw, clip)
            dh_c = jnp.minimum(dh, clip)
            pred_cx = dx * w + ctr_x
            pred_cy = dy * h + ctr_y
            pred_w = jnp.exp(dw_c) * w
            pred_h = jnp.exp(dh_c) * h
            x1 = pred_cx - 0.5 * pred_w
            y1 = pred_cy - 0.5 * pred_h
            x2 = pred_cx + 0.5 * pred_w - 1.0
            y2 = pred_cy + 0.5 * pred_h - 1.0
            k2 = win_k[pl.ds(v * L, L)]
            key = ~k2
            pos = key < 0
            bits = jnp.where(pos, key ^ SIGN, ~key)
            score = plsc.bitcast(bits, jnp.float32)
            plsc.store_scatter(out_tile, [r, c0], x1)
            plsc.store_scatter(out_tile, [r, c1], y1)
            plsc.store_scatter(out_tile, [r, c2], x2)
            plsc.store_scatter(out_tile, [r, c3], y2)
            plsc.store_scatter(out_tile, [r, c4], score)
            return carry

        lax.fori_loop(0, NV_K, dec, 0)

        pltpu.sync_copy(out_tile, out_hbm.at[pl.ds(b * K, K)])


@jax.jit
def _rpn_sc(obj_flat, anc_flat, breg_flat):
    mesh = plsc.VectorSubcoreMesh(core_axis_name="c", subcore_axis_name="s")
    kern = pl.kernel(
        _body,
        out_type=jax.ShapeDtypeStruct((N_BATCH * K, 5), jnp.float32),
        mesh=mesh,
        compiler_params=pltpu.CompilerParams(needs_layout_passes=False),
        scratch_types=[
            pltpu.VMEM((N_ANC,), jnp.float32),
            pltpu.VMEM((CAP,), jnp.int32),
            pltpu.VMEM((CAP,), jnp.int32),
            pltpu.VMEM((WCAP,), jnp.int32),
            pltpu.VMEM((WCAP,), jnp.int32),
            pltpu.VMEM((WCAP,), jnp.int32),
            pltpu.VMEM((WCAP,), jnp.int32),
            pltpu.VMEM((256,), jnp.int32),
            pltpu.VMEM((256,), jnp.int32),
            pltpu.VMEM((K,), jnp.int32),
            pltpu.VMEM((K, 4), jnp.float32),
            pltpu.VMEM((K, 4), jnp.float32),
            pltpu.VMEM((K, 5), jnp.float32),
            pltpu.SemaphoreType.DMA,
            pltpu.SemaphoreType.DMA,
        ],
    )
    return kern(obj_flat, anc_flat, breg_flat)


def kernel(anchors, objectness, box_regression):
    obj_flat = objectness.reshape(-1)
    anc_flat = anchors.reshape(-1, 4)
    breg_flat = box_regression.reshape(-1, 4)
    out = _rpn_sc(obj_flat, anc_flat, breg_flat)
    return out.reshape(N_BATCH, K, 5)


# speedup vs baseline: 1.7530x; 1.7530x over previous
import functools

import jax
import jax.numpy as jnp
from jax import lax
from jax.experimental import pallas as pl
from jax.experimental.pallas import tpu as pltpu
from jax.experimental.pallas import tpu_sc as plsc
import numpy as np

BBOX_XFORM_CLIP = float(np.log(1000.0 / 16.0))

N_BATCH = 8
N_ANC = 20000
K = 2000
L = 16
NV_ALL = N_ANC // L
NV_K = K // L
CAP = N_ANC + L
WCAP = 2048 + L
SIGN = np.int32(-2147483648)


def _zero_hist(hist):
    z = jnp.zeros((L,), jnp.int32)
    for g in range(16):
        hist[pl.ds(g * L, L)] = z


def _scalar_at(vec, lane, iota):
    return jnp.max(jnp.where(iota == lane, vec, 0))


def _popcount(mask):
    return jnp.max(plsc.all_reduce_population_count(mask))


def _find_digit(hist, k_rem, iota):
    g_sums = []
    for g in range(16):
        g_sums.append(jnp.sum(hist[pl.ds(g * L, L)]))
    suffix = [jnp.int32(0)] * 17
    for g in range(15, -1, -1):
        suffix[g] = suffix[g + 1] + g_sums[g]
    gstar = jnp.int32(0)
    ag = jnp.int32(0)
    for g in range(16):
        hit = jnp.logical_and(suffix[g] >= k_rem, suffix[g + 1] < k_rem)
        gstar = jnp.where(hit, g, gstar)
        ag = jnp.where(hit, suffix[g + 1], ag)
    hvec = hist[pl.ds(gstar * L, L)]
    rcum = plsc.cumsum(jnp.flip(hvec, 0))
    need = k_rem - ag
    j = jnp.max(plsc.all_reduce_ffs(rcum >= need))
    dloc = 15 - j
    cnt_d = _scalar_at(hvec, dloc, iota)
    rc_j = _scalar_at(rcum, j, iota)
    count_above = ag + (rc_j - cnt_d)
    return gstar * L + dloc, count_above


def _body(obj_hbm, comb_hbm, out_hbm,
          stage, keys, idxs, win_k, win_i, tmp_k, tmp_i,
          hist, offs, gidx, rows, out_tile, sem1):
    cid = lax.axis_index("c")
    sid = lax.axis_index("s")
    wid = sid * 2 + cid

    @pl.when(wid < N_BATCH)
    def _():
        b = wid
        iota = lax.iota(jnp.int32, L)
        zf = jnp.zeros((L,), jnp.int32)

        pltpu.sync_copy(obj_hbm.at[pl.ds(b * N_ANC, N_ANC)], stage)

        _zero_hist(hist)

        def h1(v, carry):
            f = stage[pl.ds(v * L, L)]
            bits = plsc.bitcast(f, jnp.int32)
            neg = bits < 0
            key = jnp.where(neg, ~bits, bits ^ SIGN)
            keys[pl.ds(v * L, L)] = key
            dig = lax.shift_right_logical(key, 24)
            cnt, last = plsc.scan_count(dig)
            plsc.addupdate_scatter(hist, [dig], cnt, mask=last)
            return carry

        lax.fori_loop(0, NV_ALL, h1, 0)

        k_rem = jnp.int32(K)
        bw = jnp.int32(0)
        n_cur = jnp.int32(N_ANC)
        for p in range(4):
            shift = 24 - 8 * p
            if p > 0:
                _zero_hist(hist)

                def hp(v, carry, shift=shift, n_cur=n_cur):
                    kv = keys[pl.ds(v * L, L)]
                    valid = (v * L + iota) < n_cur
                    dig = jnp.bitwise_and(
                        lax.shift_right_logical(kv, shift), 255)
                    cnt, last = plsc.scan_count(dig, mask=valid)
                    plsc.addupdate_scatter(hist, [dig], cnt, mask=last)
                    return carry

                lax.fori_loop(0, (n_cur + L - 1) // L, hp, 0)

            dstar, count_above = _find_digit(hist, k_rem, iota)

            def cp(v, carry, shift=shift, n_cur=n_cur, p=p, dstar=dstar):
                bw_, bs_ = carry
                kv = keys[pl.ds(v * L, L)]
                if p == 0:
                    iv = v * L + iota
                    valid_w = jnp.full((L,), True)
                else:
                    iv = idxs[pl.ds(v * L, L)]
                    valid_w = (v * L + iota) < n_cur
                dig = jnp.bitwise_and(
                    lax.shift_right_logical(kv, shift), 255).astype(jnp.int32)
                win = jnp.logical_and(dig > dstar, valid_w)
                sur = jnp.logical_and(dig == dstar, valid_w)
                plsc.store_compressed(win_k.at[pl.ds(bw_, L)], kv, mask=win)
                plsc.store_compressed(win_i.at[pl.ds(bw_, L)], iv, mask=win)
                plsc.store_compressed(keys.at[pl.ds(bs_, L)], kv, mask=sur)
                plsc.store_compressed(idxs.at[pl.ds(bs_, L)], iv, mask=sur)
                return bw_ + _popcount(win), bs_ + _popcount(sur)

            bw, n_cur = lax.fori_loop(
                0, (n_cur + L - 1) // L, cp, (bw, jnp.int32(0)))
            k_rem = k_rem - count_above

        def ap(v, carry):
            win_k[pl.ds(bw + v * L, L)] = keys[pl.ds(v * L, L)]
            win_i[pl.ds(bw + v * L, L)] = idxs[pl.ds(v * L, L)]
            return carry

        lax.fori_loop(0, (k_rem + L - 1) // L, ap, 0)

        def sort_pass(src_k, src_i, dst_k, dst_i, shift, comp_in):
            _zero_hist(hist)

            def sh(v, carry):
                kv = src_k[pl.ds(v * L, L)]
                if comp_in:
                    kv = ~kv
                dig = jnp.bitwise_and(lax.shift_right_logical(kv, shift), 255)
                cnt, last = plsc.scan_count(dig)
                plsc.addupdate_scatter(hist, [dig], cnt, mask=last)
                return carry

            lax.fori_loop(0, NV_K, sh, 0)

            carry = jnp.int32(0)
            for g in range(16):
                hv = hist[pl.ds(g * L, L)]
                c = plsc.cumsum(hv)
                offs[pl.ds(g * L, L)] = c - hv + carry
                carry = carry + jnp.max(c)

            def sp(v, carry2):
                kv = src_k[pl.ds(v * L, L)]
                if comp_in:
                    kv = ~kv
                iv = src_i[pl.ds(v * L, L)]
                dig = jnp.bitwise_and(lax.shift_right_logical(kv, shift), 255)
                cnt, last = plsc.scan_count(dig)
                base = plsc.load_gather(offs, [dig])
                addr = base + cnt - 1
                plsc.store_scatter(dst_k, [addr], kv)
                plsc.store_scatter(dst_i, [addr], iv)
                plsc.addupdate_scatter(offs, [dig], cnt, mask=last)
                return carry2

            lax.fori_loop(0, NV_K, sp, 0)

        sort_pass(win_k, win_i, tmp_k, tmp_i, 0, True)
        sort_pass(tmp_k, tmp_i, win_k, win_i, 8, False)
        sort_pass(win_k, win_i, tmp_k, tmp_i, 16, False)
        sort_pass(tmp_k, tmp_i, win_k, win_i, 24, False)

        def gi(v, carry):
            gidx[pl.ds(v * L, L)] = win_i[pl.ds(v * L, L)] + b * N_ANC
            return carry

        lax.fori_loop(0, NV_K, gi, 0)

        pltpu.async_copy(comb_hbm.at[gidx], rows, sem1).wait()

        c0 = zf
        c1 = zf + 1
        c2 = zf + 2
        c3 = zf + 3
        c4 = zf + 4
        clip = jnp.full((L,), BBOX_XFORM_CLIP, jnp.float32)

        def dec(v, carry):
            r = v * L + iota
            ax1 = plsc.load_gather(rows, [r, c0])
            ay1 = plsc.load_gather(rows, [r, c1])
            ax2 = plsc.load_gather(rows, [r, c2])
            ay2 = plsc.load_gather(rows, [r, c3])
            dx = plsc.load_gather(rows, [r, zf + 4])
            dy = plsc.load_gather(rows, [r, zf + 5])
            dw = plsc.load_gather(rows, [r, zf + 6])
            dh = plsc.load_gather(rows, [r, zf + 7])
            w = ax2 - ax1 + 1.0
            h = ay2 - ay1 + 1.0
            ctr_x = ax1 + 0.5 * w
            ctr_y = ay1 + 0.5 * h
            dw_c = jnp.minimum(dw, clip)
            dh_c = jnp.minimum(dh, clip)
            pred_cx = dx * w + ctr_x
            pred_cy = dy * h + ctr_y
            pred_w = jnp.exp(dw_c) * w
            pred_h = jnp.exp(dh_c) * h
            x1 = pred_cx - 0.5 * pred_w
            y1 = pred_cy - 0.5 * pred_h
            x2 = pred_cx + 0.5 * pred_w - 1.0
            y2 = pred_cy + 0.5 * pred_h - 1.0
            k2 = win_k[pl.ds(v * L, L)]
            key = ~k2
            pos = key < 0
            bits = jnp.where(pos, key ^ SIGN, ~key)
            score = plsc.bitcast(bits, jnp.float32)
            plsc.store_scatter(out_tile, [r, c0], x1)
            plsc.store_scatter(out_tile, [r, c1], y1)
            plsc.store_scatter(out_tile, [r, c2], x2)
            plsc.store_scatter(out_tile, [r, c3], y2)
            plsc.store_scatter(out_tile, [r, c4], score)
            return carry

        lax.fori_loop(0, NV_K, dec, 0)

        pltpu.sync_copy(out_tile, out_hbm.at[pl.ds(b * K, K)])


@jax.jit
def _rpn_sc(obj_flat, comb):
    mesh = plsc.VectorSubcoreMesh(core_axis_name="c", subcore_axis_name="s")
    kern = pl.kernel(
        _body,
        out_type=jax.ShapeDtypeStruct((N_BATCH * K, 5), jnp.float32),
        mesh=mesh,
        compiler_params=pltpu.CompilerParams(
            needs_layout_passes=False, use_tc_tiling_on_sc=False),
        scratch_types=[
            pltpu.VMEM((N_ANC,), jnp.float32),
            pltpu.VMEM((CAP,), jnp.int32),
            pltpu.VMEM((CAP,), jnp.int32),
            pltpu.VMEM((WCAP,), jnp.int32),
            pltpu.VMEM((WCAP,), jnp.int32),
            pltpu.VMEM((WCAP,), jnp.int32),
            pltpu.VMEM((WCAP,), jnp.int32),
            pltpu.VMEM((256,), jnp.int32),
            pltpu.VMEM((256,), jnp.int32),
            pltpu.VMEM((K,), jnp.int32),
            pltpu.VMEM((K, 8), jnp.float32),
            pltpu.VMEM((K, 5), jnp.float32),
            pltpu.SemaphoreType.DMA,
        ],
    )
    return kern(obj_flat, comb)


def kernel(anchors, objectness, box_regression):
    obj_flat = objectness.reshape(-1)
    comb = jnp.concatenate([anchors, box_regression], axis=-1).reshape(-1, 8)
    out = _rpn_sc(obj_flat, comb)
    return out.reshape(N_BATCH, K, 5)


# speedup vs baseline: 1.7531x; 1.0000x over previous
import functools

import jax
import jax.numpy as jnp
from jax import lax
from jax.experimental import pallas as pl
from jax.experimental.pallas import tpu as pltpu
from jax.experimental.pallas import tpu_sc as plsc
import numpy as np

BBOX_XFORM_CLIP = float(np.log(1000.0 / 16.0))

N_BATCH = 8
N_ANC = 20000
K = 2000
L = 16
NV_ALL = N_ANC // L
NV_K = K // L
CAP = N_ANC + L
WCAP = 2048 + L
SIGN = np.int32(-2147483648)
UNROLL = 5


def _zero_hist(hist):
    z = jnp.zeros((L,), jnp.int32)
    for g in range(16):
        hist[pl.ds(g * L, L)] = z


def _scalar_at(vec, lane, iota):
    return jnp.max(jnp.where(iota == lane, vec, 0))


def _popcount(mask):
    return jnp.max(plsc.all_reduce_population_count(mask))


def _find_digit(hist, k_rem, iota):
    g_sums = []
    for g in range(16):
        g_sums.append(jnp.sum(hist[pl.ds(g * L, L)]))
    suffix = [jnp.int32(0)] * 17
    for g in range(15, -1, -1):
        suffix[g] = suffix[g + 1] + g_sums[g]
    gstar = jnp.int32(0)
    ag = jnp.int32(0)
    for g in range(16):
        hit = jnp.logical_and(suffix[g] >= k_rem, suffix[g + 1] < k_rem)
        gstar = jnp.where(hit, g, gstar)
        ag = jnp.where(hit, suffix[g + 1], ag)
    hvec = hist[pl.ds(gstar * L, L)]
    rcum = plsc.cumsum(jnp.flip(hvec, 0))
    need = k_rem - ag
    j = jnp.max(plsc.all_reduce_ffs(rcum >= need))
    dloc = 15 - j
    cnt_d = _scalar_at(hvec, dloc, iota)
    rc_j = _scalar_at(rcum, j, iota)
    count_above = ag + (rc_j - cnt_d)
    return gstar * L + dloc, count_above


def _body(obj_hbm, comb_hbm, out_hbm,
          stage, keys, idxs, win_k, win_i, tmp_k, tmp_i,
          hist, offs, gidx, rows, out_tile, sem1):
    cid = lax.axis_index("c")
    sid = lax.axis_index("s")
    wid = sid * 2 + cid

    @pl.when(wid < N_BATCH)
    def _():
        b = wid
        iota = lax.iota(jnp.int32, L)
        zf = jnp.zeros((L,), jnp.int32)

        pltpu.sync_copy(obj_hbm.at[pl.ds(b * N_ANC, N_ANC)], stage)

        _zero_hist(hist)

        def h1(v, carry):
            for j in range(UNROLL):
                u = v * UNROLL + j
                f = stage[pl.ds(u * L, L)]
                bits = plsc.bitcast(f, jnp.int32)
                neg = bits < 0
                key = jnp.where(neg, ~bits, bits ^ SIGN)
                keys[pl.ds(u * L, L)] = key
                dig = lax.shift_right_logical(key, 24)
                cnt, last = plsc.scan_count(dig)
                plsc.addupdate_scatter(hist, [dig], cnt, mask=last)
            return carry

        lax.fori_loop(0, NV_ALL // UNROLL, h1, 0)

        k_rem = jnp.int32(K)
        bw = jnp.int32(0)
        n_cur = jnp.int32(N_ANC)
        for p in range(4):
            shift = 24 - 8 * p
            if p > 0:
                _zero_hist(hist)

                def hp(v, carry, shift=shift, n_cur=n_cur):
                    kv = keys[pl.ds(v * L, L)]
                    valid = (v * L + iota) < n_cur
                    dig = jnp.bitwise_and(
                        lax.shift_right_logical(kv, shift), 255)
                    cnt, last = plsc.scan_count(dig, mask=valid)
                    plsc.addupdate_scatter(hist, [dig], cnt, mask=last)
                    return carry

                lax.fori_loop(0, (n_cur + L - 1) // L, hp, 0)

            dstar, count_above = _find_digit(hist, k_rem, iota)

            def cp_one(u, bw_, bs_, shift=shift, n_cur=n_cur, p=p, dstar=dstar):
                kv = keys[pl.ds(u * L, L)]
                if p == 0:
                    iv = u * L + iota
                    valid_w = jnp.full((L,), True)
                else:
                    iv = idxs[pl.ds(u * L, L)]
                    valid_w = (u * L + iota) < n_cur
                dig = jnp.bitwise_and(
                    lax.shift_right_logical(kv, shift), 255).astype(jnp.int32)
                win = jnp.logical_and(dig > dstar, valid_w)
                sur = jnp.logical_and(dig == dstar, valid_w)
                plsc.store_compressed(win_k.at[pl.ds(bw_, L)], kv, mask=win)
                plsc.store_compressed(win_i.at[pl.ds(bw_, L)], iv, mask=win)
                plsc.store_compressed(keys.at[pl.ds(bs_, L)], kv, mask=sur)
                plsc.store_compressed(idxs.at[pl.ds(bs_, L)], iv, mask=sur)
                return bw_ + _popcount(win), bs_ + _popcount(sur)

            if p == 0:
                def cp(v, carry):
                    bw_, bs_ = carry
                    for j in range(UNROLL):
                        bw_, bs_ = cp_one(v * UNROLL + j, bw_, bs_)
                    return bw_, bs_

                bw, n_cur = lax.fori_loop(
                    0, NV_ALL // UNROLL, cp, (bw, jnp.int32(0)))
            else:
                def cp(v, carry):
                    return cp_one(v, *carry)

                bw, n_cur = lax.fori_loop(
                    0, (n_cur + L - 1) // L, cp, (bw, jnp.int32(0)))
            k_rem = k_rem - count_above

        def ap(v, carry):
            win_k[pl.ds(bw + v * L, L)] = keys[pl.ds(v * L, L)]
            win_i[pl.ds(bw + v * L, L)] = idxs[pl.ds(v * L, L)]
            return carry

        lax.fori_loop(0, (k_rem + L - 1) // L, ap, 0)

        def sort_pass(src_k, src_i, dst_k, dst_i, shift, comp_in):
            _zero_hist(hist)

            def sh(v, carry):
                for j in range(UNROLL):
                    u = v * UNROLL + j
                    kv = src_k[pl.ds(u * L, L)]
                    if comp_in:
                        kv = ~kv
                    dig = jnp.bitwise_and(
                        lax.shift_right_logical(kv, shift), 255)
                    cnt, last = plsc.scan_count(dig)
                    plsc.addupdate_scatter(hist, [dig], cnt, mask=last)
                return carry

            lax.fori_loop(0, NV_K // UNROLL, sh, 0)

            carry = jnp.int32(0)
            for g in range(16):
                hv = hist[pl.ds(g * L, L)]
                c = plsc.cumsum(hv)
                offs[pl.ds(g * L, L)] = c - hv + carry
                carry = carry + jnp.max(c)

            def sp(v, carry2):
                for j in range(UNROLL):
                    u = v * UNROLL + j
                    kv = src_k[pl.ds(u * L, L)]
                    if comp_in:
                        kv = ~kv
                    iv = src_i[pl.ds(u * L, L)]
                    dig = jnp.bitwise_and(
                        lax.shift_right_logical(kv, shift), 255)
                    cnt, last = plsc.scan_count(dig)
                    base = plsc.load_gather(offs, [dig])
                    addr = base + cnt - 1
                    plsc.store_scatter(dst_k, [addr], kv)
                    plsc.store_scatter(dst_i, [addr], iv)
                    plsc.addupdate_scatter(offs, [dig], cnt, mask=last)
                return carry2

            lax.fori_loop(0, NV_K // UNROLL, sp, 0)

        sort_pass(win_k, win_i, tmp_k, tmp_i, 0, True)
        sort_pass(tmp_k, tmp_i, win_k, win_i, 8, False)
        sort_pass(win_k, win_i, tmp_k, tmp_i, 16, False)
        sort_pass(tmp_k, tmp_i, win_k, win_i, 24, False)

        def gi(v, carry):
            for j in range(UNROLL):
                u = v * UNROLL + j
                gidx[pl.ds(u * L, L)] = win_i[pl.ds(u * L, L)] + b * N_ANC
            return carry

        lax.fori_loop(0, NV_K // UNROLL, gi, 0)

        pltpu.async_copy(comb_hbm.at[gidx], rows, sem1).wait()

        c0 = zf
        c1 = zf + 1
        c2 = zf + 2
        c3 = zf + 3
        c4 = zf + 4
        clip = jnp.full((L,), BBOX_XFORM_CLIP, jnp.float32)

        def dec_one(u):
            r = u * L + iota
            ax1 = plsc.load_gather(rows, [r, c0])
            ay1 = plsc.load_gather(rows, [r, c1])
            ax2 = plsc.load_gather(rows, [r, c2])
            ay2 = plsc.load_gather(rows, [r, c3])
            dx = plsc.load_gather(rows, [r, zf + 4])
            dy = plsc.load_gather(rows, [r, zf + 5])
            dw = plsc.load_gather(rows, [r, zf + 6])
            dh = plsc.load_gather(rows, [r, zf + 7])
            w = ax2 - ax1 + 1.0
            h = ay2 - ay1 + 1.0
            ctr_x = ax1 + 0.5 * w
            ctr_y = ay1 + 0.5 * h
            dw_c = jnp.minimum(dw, clip)
            dh_c = jnp.minimum(dh, clip)
            pred_cx = dx * w + ctr_x
            pred_cy = dy * h + ctr_y
            pred_w = jnp.exp(dw_c) * w
            pred_h = jnp.exp(dh_c) * h
            x1 = pred_cx - 0.5 * pred_w
            y1 = pred_cy - 0.5 * pred_h
            x2 = pred_cx + 0.5 * pred_w - 1.0
            y2 = pred_cy + 0.5 * pred_h - 1.0
            k2 = win_k[pl.ds(u * L, L)]
            key = ~k2
            pos = key < 0
            bits = jnp.where(pos, key ^ SIGN, ~key)
            score = plsc.bitcast(bits, jnp.float32)
            plsc.store_scatter(out_tile, [r, c0], x1)
            plsc.store_scatter(out_tile, [r, c1], y1)
            plsc.store_scatter(out_tile, [r, c2], x2)
            plsc.store_scatter(out_tile, [r, c3], y2)
            plsc.store_scatter(out_tile, [r, c4], score)

        def dec(v, carry):
            for j in range(UNROLL):
                dec_one(v * UNROLL + j)
            return carry

        lax.fori_loop(0, NV_K // UNROLL, dec, 0)

        pltpu.sync_copy(out_tile, out_hbm.at[pl.ds(b * K, K)])


@jax.jit
def _rpn_sc(obj_flat, comb):
    mesh = plsc.VectorSubcoreMesh(core_axis_name="c", subcore_axis_name="s")
    kern = pl.kernel(
        _body,
        out_type=jax.ShapeDtypeStruct((N_BATCH * K, 5), jnp.float32),
        mesh=mesh,
        compiler_params=pltpu.CompilerParams(
            needs_layout_passes=False, use_tc_tiling_on_sc=False),
        scratch_types=[
            pltpu.VMEM((N_ANC,), jnp.float32),
            pltpu.VMEM((CAP,), jnp.int32),
            pltpu.VMEM((CAP,), jnp.int32),
            pltpu.VMEM((WCAP,), jnp.int32),
            pltpu.VMEM((WCAP,), jnp.int32),
            pltpu.VMEM((WCAP,), jnp.int32),
            pltpu.VMEM((WCAP,), jnp.int32),
            pltpu.VMEM((256,), jnp.int32),
            pltpu.VMEM((256,), jnp.int32),
            pltpu.VMEM((K,), jnp.int32),
            pltpu.VMEM((K, 8), jnp.float32),
            pltpu.VMEM((K, 5), jnp.float32),
            pltpu.SemaphoreType.DMA,
        ],
    )
    return kern(obj_flat, comb)


def kernel(anchors, objectness, box_regression):
    obj_flat = objectness.reshape(-1)
    comb = jnp.concatenate([anchors, box_regression], axis=-1).reshape(-1, 8)
    out = _rpn_sc(obj_flat, comb)
    return out.reshape(N_BATCH, K, 5)
